# baseline (speedup 1.0000x reference)
import jax
import jax.numpy as jnp
from jax.experimental import pallas as pl
from jax.experimental.pallas import tpu as pltpu

CAP = 100000
FDIM = 128
BATCH = 4096
ROWS = 5000
NBLK = CAP // ROWS
PAD = BATCH + 2 * ROWS

LLANE = 125
LROWS_TOT = CAP // LLANE
LBLK = LROWS_TOT // NBLK
LSRC = (LLANE + BATCH + LLANE - 1) // LLANE
LPADTOP = LBLK
LSRC_PAD = -(-(LSRC + 2 * LBLK) // 8) * 8


def _enqueue_kernel(scal_ref, vl_ref, ls_ref, q_ref, ql_ref, oq_ref, ol_ref):
    b = pl.program_id(0)
    s = b * ROWS
    p = scal_ref[0]

    c0 = s - p
    c0 = jnp.where(c0 < 0, c0 + CAP, c0)
    has = (c0 < BATCH) | (c0 >= CAP - ROWS)

    @pl.when(has)
    def _():
        rows = jax.lax.broadcasted_iota(jnp.int32, (ROWS, 1), 0) + s
        m = rows - p
        m = jnp.where(m < 0, m + CAP, m)
        in_win = m < BATCH
        c = jnp.where(c0 >= CAP - ROWS, c0 - CAP, c0)
        o = jnp.clip(c + ROWS, 0, BATCH + ROWS)
        oq_ref[...] = jnp.where(in_win, vl_ref[pl.ds(o, ROWS), :], q_ref[...])

    @pl.when(jnp.logical_not(has))
    def _():
        oq_ref[...] = q_ref[...]

    rowoff = scal_ref[1]
    li = jax.lax.broadcasted_iota(jnp.int32, (LBLK, LLANE), 0) + b * LBLK
    lj = jax.lax.broadcasted_iota(jnp.int32, (LBLK, LLANE), 1)
    k = li * LLANE + lj
    mk = k - p
    mk = jnp.where(mk < 0, mk + CAP, mk)
    lwin = mk < BATCH
    t = b * LBLK - rowoff
    t = jnp.where(t < 0, t + LROWS_TOT, t)
    cl = jnp.where(t >= LROWS_TOT - LBLK, t - LROWS_TOT, t)
    ol = jnp.clip(cl + LPADTOP, 0, LSRC + LBLK)
    ol_ref[...] = jnp.where(lwin, ls_ref[pl.ds(ol, LBLK), :], ql_ref[...])


def _enqueue(experience_queue, ql2d, vl_feat, lsrc2d, scal):
    grid_spec = pltpu.PrefetchScalarGridSpec(
        num_scalar_prefetch=1,
        grid=(NBLK,),
        in_specs=[
            pl.BlockSpec((PAD, FDIM), lambda b, sp: (0, 0)),
            pl.BlockSpec((LSRC_PAD, LLANE), lambda b, sp: (0, 0)),
            pl.BlockSpec((ROWS, FDIM), lambda b, sp: (b, 0)),
            pl.BlockSpec((LBLK, LLANE), lambda b, sp: (b, 0)),
        ],
        out_specs=[
            pl.BlockSpec((ROWS, FDIM), lambda b, sp: (b, 0)),
            pl.BlockSpec((LBLK, LLANE), lambda b, sp: (b, 0)),
        ],
    )
    vl_pad = jnp.pad(vl_feat, ((ROWS, ROWS), (0, 0)))
    return pl.pallas_call(
        _enqueue_kernel,
        grid_spec=grid_spec,
        compiler_params=pltpu.CompilerParams(
            dimension_semantics=("parallel",),
        ),
        out_shape=[
            jax.ShapeDtypeStruct((CAP, FDIM), jnp.float32),
            jax.ShapeDtypeStruct((LROWS_TOT, LLANE), jnp.float32),
        ],
    )(scal, vl_pad, lsrc2d, experience_queue, ql2d)


def kernel(experience_queue, exp_label_queue, vl_feat, label, ptr):
    p = jnp.asarray(ptr, dtype=jnp.int32)
    q_ = p % LLANE
    rowoff = (p - q_) // LLANE
    s_flat = jax.lax.dynamic_update_slice(
        jnp.zeros((LSRC * LLANE,), jnp.float32), label.reshape(BATCH), (q_,)
    )
    lsrc2d = jnp.pad(
        s_flat.reshape(LSRC, LLANE),
        ((LPADTOP, LSRC_PAD - LSRC - LPADTOP), (0, 0)),
    )
    ql2d = exp_label_queue.reshape(LROWS_TOT, LLANE)
    scal = jnp.stack([p, rowoff])
    new_queue, nl2d = _enqueue(experience_queue, ql2d, vl_feat, lsrc2d, scal)
    new_labels = nl2d.reshape(CAP, 1)
    new_ptr = (p + BATCH) % CAP
    is_full = jnp.where(new_ptr < p, 1, 0).astype(jnp.int64)
    is_empty = jnp.where(BATCH > 0, 0, 1).astype(jnp.int64)
    return new_queue, new_labels, jnp.asarray(new_ptr, dtype=jnp.int64), is_full, is_empty


# speedup vs baseline: 1.1220x; 1.1220x over previous
import jax
import jax.numpy as jnp
from jax.experimental import pallas as pl
from jax.experimental.pallas import tpu as pltpu

CAP = 100000
FDIM = 128
BATCH = 4096
ROWS = 5000
NBLK = CAP // ROWS
PAD = BATCH + 2 * ROWS

LLANE = 125
LROWS_TOT = CAP // LLANE
LBLK = LROWS_TOT // NBLK
LSRC = (LLANE + BATCH + LLANE - 1) // LLANE
LPADTOP = LBLK
LSRC_PAD = -(-(LSRC + 2 * LBLK) // 8) * 8


def _enqueue_kernel(scal_ref, vl_ref, ls_ref, q_ref, ql_ref, oq_ref, ol_ref, vs_ref):
    b = pl.program_id(0)
    s = b * ROWS
    p = scal_ref[0]

    @pl.when(b == 0)
    def _():
        vs_ref[pl.ds(ROWS, BATCH), :] = vl_ref[...]

    c0 = s - p
    c0 = jnp.where(c0 < 0, c0 + CAP, c0)
    has = (c0 < BATCH) | (c0 >= CAP - ROWS)

    @pl.when(has)
    def _():
        rows = jax.lax.broadcasted_iota(jnp.int32, (ROWS, 1), 0) + s
        m = rows - p
        m = jnp.where(m < 0, m + CAP, m)
        in_win = m < BATCH
        c = jnp.where(c0 >= CAP - ROWS, c0 - CAP, c0)
        o = jnp.clip(c + ROWS, 0, BATCH + ROWS)
        oq_ref[...] = jnp.where(in_win, vs_ref[pl.ds(o, ROWS), :], q_ref[...])

    @pl.when(jnp.logical_not(has))
    def _():
        oq_ref[...] = q_ref[...]

    rowoff = scal_ref[1]
    li = jax.lax.broadcasted_iota(jnp.int32, (LBLK, LLANE), 0) + b * LBLK
    lj = jax.lax.broadcasted_iota(jnp.int32, (LBLK, LLANE), 1)
    k = li * LLANE + lj
    mk = k - p
    mk = jnp.where(mk < 0, mk + CAP, mk)
    lwin = mk < BATCH
    t = b * LBLK - rowoff
    t = jnp.where(t < 0, t + LROWS_TOT, t)
    cl = jnp.where(t >= LROWS_TOT - LBLK, t - LROWS_TOT, t)
    ol = jnp.clip(cl + LPADTOP, 0, LSRC + LBLK)
    ol_ref[...] = jnp.where(lwin, ls_ref[pl.ds(ol, LBLK), :], ql_ref[...])


def _enqueue(experience_queue, ql2d, vl_feat, lsrc2d, scal):
    grid_spec = pltpu.PrefetchScalarGridSpec(
        num_scalar_prefetch=1,
        grid=(NBLK,),
        in_specs=[
            pl.BlockSpec((BATCH, FDIM), lambda b, sp: (0, 0)),
            pl.BlockSpec((LSRC_PAD, LLANE), lambda b, sp: (0, 0)),
            pl.BlockSpec((ROWS, FDIM), lambda b, sp: (b, 0)),
            pl.BlockSpec((LBLK, LLANE), lambda b, sp: (b, 0)),
        ],
        out_specs=[
            pl.BlockSpec((ROWS, FDIM), lambda b, sp: (b, 0)),
            pl.BlockSpec((LBLK, LLANE), lambda b, sp: (b, 0)),
        ],
        scratch_shapes=[pltpu.VMEM((PAD, FDIM), jnp.float32)],
    )
    return pl.pallas_call(
        _enqueue_kernel,
        grid_spec=grid_spec,
        compiler_params=pltpu.CompilerParams(
            dimension_semantics=("arbitrary",),
        ),
        out_shape=[
            jax.ShapeDtypeStruct((CAP, FDIM), jnp.float32),
            jax.ShapeDtypeStruct((LROWS_TOT, LLANE), jnp.float32),
        ],
    )(scal, vl_feat, lsrc2d, experience_queue, ql2d)


def kernel(experience_queue, exp_label_queue, vl_feat, label, ptr):
    p = jnp.asarray(ptr, dtype=jnp.int32)
    q_ = p % LLANE
    rowoff = (p - q_) // LLANE
    s_flat = jax.lax.dynamic_update_slice(
        jnp.zeros((LSRC * LLANE,), jnp.float32), label.reshape(BATCH), (q_,)
    )
    lsrc2d = jnp.pad(
        s_flat.reshape(LSRC, LLANE),
        ((LPADTOP, LSRC_PAD - LSRC - LPADTOP), (0, 0)),
    )
    ql2d = exp_label_queue.reshape(LROWS_TOT, LLANE)
    scal = jnp.stack([p, rowoff])
    new_queue, nl2d = _enqueue(experience_queue, ql2d, vl_feat, lsrc2d, scal)
    new_labels = nl2d.reshape(CAP, 1)
    new_ptr = (p + BATCH) % CAP
    is_full = jnp.where(new_ptr < p, 1, 0).astype(jnp.int64)
    is_empty = jnp.where(BATCH > 0, 0, 1).astype(jnp.int64)
    return new_queue, new_labels, jnp.asarray(new_ptr, dtype=jnp.int64), is_full, is_empty


# speedup vs baseline: 1.1360x; 1.0125x over previous
import jax
import jax.numpy as jnp
from jax.experimental import pallas as pl
from jax.experimental.pallas import tpu as pltpu

CAP = 100000
FDIM = 128
BATCH = 4096
ROWS = 10000
NBLK = CAP // ROWS
PAD = BATCH + 2 * ROWS

LLANE = 125
LROWS_TOT = CAP // LLANE
LBLK = LROWS_TOT // NBLK
LSRC = (LLANE + BATCH + LLANE - 1) // LLANE
LPADTOP = LBLK
LSRC_PAD = -(-(LSRC + 2 * LBLK) // 8) * 8


def _enqueue_kernel(scal_ref, vl_ref, ls_ref, q_ref, ql_ref, oq_ref, ol_ref, vs_ref):
    b = pl.program_id(0)
    s = b * ROWS
    p = scal_ref[0]

    @pl.when(b == 0)
    def _():
        vs_ref[pl.ds(ROWS, BATCH), :] = vl_ref[...]

    c0 = s - p
    c0 = jnp.where(c0 < 0, c0 + CAP, c0)
    has = (c0 < BATCH) | (c0 >= CAP - ROWS)

    @pl.when(has)
    def _():
        rows = jax.lax.broadcasted_iota(jnp.int32, (ROWS, 1), 0) + s
        m = rows - p
        m = jnp.where(m < 0, m + CAP, m)
        in_win = m < BATCH
        c = jnp.where(c0 >= CAP - ROWS, c0 - CAP, c0)
        o = jnp.clip(c + ROWS, 0, BATCH + ROWS)
        oq_ref[...] = jnp.where(in_win, vs_ref[pl.ds(o, ROWS), :], q_ref[...])

    @pl.when(jnp.logical_not(has))
    def _():
        oq_ref[...] = q_ref[...]

    rowoff = scal_ref[1]
    li = jax.lax.broadcasted_iota(jnp.int32, (LBLK, LLANE), 0) + b * LBLK
    lj = jax.lax.broadcasted_iota(jnp.int32, (LBLK, LLANE), 1)
    k = li * LLANE + lj
    mk = k - p
    mk = jnp.where(mk < 0, mk + CAP, mk)
    lwin = mk < BATCH
    t = b * LBLK - rowoff
    t = jnp.where(t < 0, t + LROWS_TOT, t)
    cl = jnp.where(t >= LROWS_TOT - LBLK, t - LROWS_TOT, t)
    ol = jnp.clip(cl + LPADTOP, 0, LSRC + LBLK)
    ol_ref[...] = jnp.where(lwin, ls_ref[pl.ds(ol, LBLK), :], ql_ref[...])


def _enqueue(experience_queue, ql2d, vl_feat, lsrc2d, scal):
    grid_spec = pltpu.PrefetchScalarGridSpec(
        num_scalar_prefetch=1,
        grid=(NBLK,),
        in_specs=[
            pl.BlockSpec((BATCH, FDIM), lambda b, sp: (0, 0)),
            pl.BlockSpec((LSRC_PAD, LLANE), lambda b, sp: (0, 0)),
            pl.BlockSpec((ROWS, FDIM), lambda b, sp: (b, 0)),
            pl.BlockSpec((LBLK, LLANE), lambda b, sp: (b, 0)),
        ],
        out_specs=[
            pl.BlockSpec((ROWS, FDIM), lambda b, sp: (b, 0)),
            pl.BlockSpec((LBLK, LLANE), lambda b, sp: (b, 0)),
        ],
        scratch_shapes=[pltpu.VMEM((PAD, FDIM), jnp.float32)],
    )
    return pl.pallas_call(
        _enqueue_kernel,
        grid_spec=grid_spec,
        compiler_params=pltpu.CompilerParams(
            dimension_semantics=("arbitrary",),
        ),
        out_shape=[
            jax.ShapeDtypeStruct((CAP, FDIM), jnp.float32),
            jax.ShapeDtypeStruct((LROWS_TOT, LLANE), jnp.float32),
        ],
    )(scal, vl_feat, lsrc2d, experience_queue, ql2d)


def kernel(experience_queue, exp_label_queue, vl_feat, label, ptr):
    p = jnp.asarray(ptr, dtype=jnp.int32)
    q_ = p % LLANE
    rowoff = (p - q_) // LLANE
    s_flat = jax.lax.dynamic_update_slice(
        jnp.zeros((LSRC * LLANE,), jnp.float32), label.reshape(BATCH), (q_,)
    )
    lsrc2d = jnp.pad(
        s_flat.reshape(LSRC, LLANE),
        ((LPADTOP, LSRC_PAD - LSRC - LPADTOP), (0, 0)),
    )
    ql2d = exp_label_queue.reshape(LROWS_TOT, LLANE)
    scal = jnp.stack([p, rowoff])
    new_queue, nl2d = _enqueue(experience_queue, ql2d, vl_feat, lsrc2d, scal)
    new_labels = nl2d.reshape(CAP, 1)
    new_ptr = (p + BATCH) % CAP
    is_full = jnp.where(new_ptr < p, 1, 0).astype(jnp.int64)
    is_empty = jnp.where(BATCH > 0, 0, 1).astype(jnp.int64)
    return new_queue, new_labels, jnp.asarray(new_ptr, dtype=jnp.int64), is_full, is_empty
